# baseline (speedup 1.0000x reference)
import functools

import jax
import jax.numpy as jnp
from jax import lax
from jax.experimental import pallas as pl
from jax.experimental.pallas import tpu as pltpu
from jax.experimental.pallas import tpu_sc as plsc

N = 10000
E = 320000
NUM_GRAPHS = 64
HALF = 128
NC, NS = 2, 16
EK = 128
ECHUNKS = E // EK
NK = 80
NCHUNKS = N // NK
RT0 = 624
RTC = 640

_mesh = plsc.VectorSubcoreMesh(
    core_axis_name="c", subcore_axis_name="s", num_cores=NC, num_subcores=NS)


DT = 80


@functools.partial(
    pl.kernel,
    out_type=(
        jax.ShapeDtypeStruct((NC, N, HALF), jnp.float32),
        jax.ShapeDtypeStruct((NUM_GRAPHS, HALF), jnp.float32),
    ),
    mesh=_mesh,
    scratch_types=[
        pltpu.VMEM_SHARED((N, HALF), jnp.float32),
        pltpu.VMEM_SHARED((NUM_GRAPHS, HALF), jnp.float32),
        pltpu.VMEM((EK, HALF), jnp.float32),
        pltpu.VMEM((NK, HALF), jnp.float32),
        pltpu.VMEM((DT, EK), jnp.int32),
        pltpu.VMEM((NK,), jnp.int32),
        pltpu.SemaphoreType.DMA,
        pltpu.SemaphoreType.DMA,
    ],
)
def _sc_deg_cnt(dstdeg_t, batch3d, ones_in, zeros_in, deg_out, cnt_out,
                deg_s, cnt_s, ones_e, ones_n, idx_e, idx_n, semA, semB):
    c = lax.axis_index("c")
    s = lax.axis_index("s")
    t = c * NS + s
    pltpu.sync_copy(ones_in, ones_e)
    nch = jnp.clip(ECHUNKS - t * DT, 0, DT)
    pltpu.sync_copy(dstdeg_t.at[t], idx_e)
    pltpu.sync_copy(zeros_in, deg_s.at[pl.ds(s * RT0, RTC)])

    @pl.when((c == 1) & (s == 0))
    def _():
        pltpu.sync_copy(zeros_in.at[pl.ds(0, NUM_GRAPHS)], cnt_s)

    plsc.subcore_barrier()

    def body(m, carry):
        kA = 2 * m
        dA = pltpu.async_copy(ones_e, deg_s.at[idx_e.at[kA]], semA, add=True)
        kB = kA + 1

        @pl.when(kB < nch)
        def _():
            dB = pltpu.async_copy(ones_e, deg_s.at[idx_e.at[kB]], semB,
                                  add=True)
            dA.wait()
            dB.wait()

        @pl.when(kB >= nch)
        def _():
            dA.wait()

        return carry

    lax.fori_loop(0, (nch + 1) // 2, body, 0)

    @pl.when(c == 1)
    def _():
        pltpu.sync_copy(ones_in.at[pl.ds(0, NK)], ones_n)
        nchn = (NCHUNKS - s + NS - 1) // NS

        def bodyn(k, carry):
            i = s + NS * k
            pltpu.sync_copy(batch3d.at[i, 0], idx_n)
            pltpu.sync_copy(ones_n, cnt_s.at[idx_n], add=True)
            return carry

        lax.fori_loop(0, nchn, bodyn, 0)

    plsc.subcore_barrier()
    pltpu.sync_copy(deg_s.at[pl.ds(s * RT0, RTC)],
                    deg_out.at[c].at[pl.ds(s * RT0, RTC)])

    @pl.when((c == 1) & (s == 0))
    def _():
        pltpu.sync_copy(cnt_s, cnt_out)


SEG = 32
NSEG = 5
CTP = SEG * NSEG


@functools.partial(
    pl.kernel,
    out_type=jax.ShapeDtypeStruct((NC, N, HALF), jnp.float32),
    mesh=_mesh,
    scratch_types=[
        pltpu.VMEM_SHARED((N, HALF), jnp.float32),
        pltpu.VMEM((EK, HALF), jnp.float32),
        pltpu.VMEM((EK, HALF), jnp.float32),
        pltpu.VMEM((SEG, EK), jnp.int32),
        pltpu.VMEM((SEG, EK), jnp.int32),
        pltpu.SemaphoreType.DMA,
        pltpu.SemaphoreType.DMA,
        pltpu.SemaphoreType.DMA,
        pltpu.SemaphoreType.DMA,
    ],
)
def _sc_conv(hs, src_t, dst_t, out, acc, rowsA, rowsB, isrc, idst,
             semA, semB, semSA, semSB):
    c = lax.axis_index("c")
    s = lax.axis_index("s")
    hs_c = hs.at[c]
    nch = jnp.clip(ECHUNKS - s * CTP, 0, CTP)
    pltpu.sync_copy(hs_c.at[pl.ds(s * RT0, RTC)], acc.at[pl.ds(s * RT0, RTC)])
    plsc.subcore_barrier()

    def seg_body(g, carry0):
        nseg = jnp.clip(nch - g * SEG, 0, SEG)

        @pl.when(nseg > 0)
        def _():
            pltpu.sync_copy(src_t.at[s].at[pl.ds(g * SEG, SEG)], isrc)
            pltpu.sync_copy(dst_t.at[s].at[pl.ds(g * SEG, SEG)], idst)

            def body(m, carry):
                kA = 2 * m
                kB = kA + 1

                @pl.when(kA >= 2)
                def _():
                    pltpu.make_async_copy(hs_c.at[pl.ds(0, EK)], rowsA,
                                          semSA).wait()

                gA = pltpu.async_copy(hs_c.at[isrc.at[kA]], rowsA, semA)

                @pl.when(kB >= 2)
                def _():
                    pltpu.make_async_copy(hs_c.at[pl.ds(0, EK)], rowsB,
                                          semSB).wait()

                @pl.when(kB < nseg)
                def _():
                    gB = pltpu.async_copy(hs_c.at[isrc.at[kB]], rowsB, semB)
                    gA.wait()
                    pltpu.async_copy(rowsA, acc.at[idst.at[kA]], semSA,
                                     add=True)
                    gB.wait()
                    pltpu.async_copy(rowsB, acc.at[idst.at[kB]], semSB,
                                     add=True)

                @pl.when(kB >= nseg)
                def _():
                    gA.wait()
                    pltpu.async_copy(rowsA, acc.at[idst.at[kA]], semSA,
                                     add=True)

                return carry

            lax.fori_loop(0, (nseg + 1) // 2, body, 0)
            pltpu.make_async_copy(hs_c.at[pl.ds(0, EK)], rowsA, semSA).wait()

            @pl.when((nseg % 2 == 0) & (nseg >= 2))
            def _():
                pltpu.make_async_copy(hs_c.at[pl.ds(0, EK)], rowsB,
                                      semSB).wait()

        return carry0

    lax.fori_loop(0, NSEG, seg_body, 0)
    plsc.subcore_barrier()
    pltpu.sync_copy(acc.at[pl.ds(s * RT0, RTC)],
                    out.at[c].at[pl.ds(s * RT0, RTC)])


@functools.partial(
    pl.kernel,
    out_type=jax.ShapeDtypeStruct((NC, NUM_GRAPHS, HALF), jnp.float32),
    mesh=_mesh,
    scratch_types=[
        pltpu.VMEM_SHARED((NUM_GRAPHS, HALF), jnp.float32),
        pltpu.VMEM((NK, HALF), jnp.float32),
        pltpu.VMEM((NK,), jnp.int32),
    ],
)
def _sc_pool(h3, batch3d, zeros_in, out, sums_s, rows, idx):
    c = lax.axis_index("c")
    s = lax.axis_index("s")
    h3_c = h3.at[c]

    @pl.when(s == 0)
    def _():
        pltpu.sync_copy(zeros_in, sums_s)

    plsc.subcore_barrier()
    nch = (NCHUNKS - s + NS - 1) // NS

    def body(k, carry):
        i = s + NS * k
        pltpu.sync_copy(batch3d.at[i, 0], idx)
        pltpu.sync_copy(h3_c.at[pl.ds(i * NK, NK)], rows)
        pltpu.sync_copy(rows, sums_s.at[idx], add=True)
        return carry

    lax.fori_loop(0, nch, body, 0)
    plsc.subcore_barrier()

    @pl.when(s == 0)
    def _():
        pltpu.sync_copy(sums_s, out.at[c])


_RB = 1000


def _tc_l1_body(x_ref, w_ref, deg_ref, out_ref, dinv_ref):
    deg = deg_ref[0, :, 0:1] + deg_ref[1, :, 0:1]
    dinv = lax.rsqrt(deg + 1.0)
    dinv_ref[...] = jnp.broadcast_to(dinv, (_RB, 16))
    h = jnp.dot(x_ref[...], w_ref[...], preferred_element_type=jnp.float32)
    out_ref[0] = dinv * h


def _tc_l1(x, W1, deg_parts):
    return pl.pallas_call(
        _tc_l1_body,
        grid=(2, N // _RB),
        in_specs=[
            pl.BlockSpec((_RB, 128), lambda j, r: (r, 0)),
            pl.BlockSpec((128, HALF), lambda j, r: (0, j)),
            pl.BlockSpec((NC, _RB, HALF), lambda j, r: (0, r, 0)),
        ],
        out_specs=[
            pl.BlockSpec((1, _RB, HALF), lambda j, r: (j, r, 0)),
            pl.BlockSpec((_RB, 16), lambda j, r: (r, 0)),
        ],
        out_shape=[
            jax.ShapeDtypeStruct((NC, N, HALF), jnp.float32),
            jax.ShapeDtypeStruct((N, 16), jnp.float32),
        ],
    )(x, W1, deg_parts)


def _tc_layer_body(s_ref, w_ref, dinv_ref, b_ref, out_ref):
    dinv = dinv_ref[:, 0:1]
    agg = jnp.concatenate([s_ref[0], s_ref[1]], axis=-1)
    a = jnp.maximum(dinv * agg + b_ref[...], 0.0)
    h = jnp.dot(a, w_ref[...], preferred_element_type=jnp.float32)
    out_ref[0] = dinv * h


def _tc_layer(s, W, b2d, dinv16):
    return pl.pallas_call(
        _tc_layer_body,
        grid=(2, N // _RB),
        in_specs=[
            pl.BlockSpec((NC, _RB, HALF), lambda j, r: (0, r, 0)),
            pl.BlockSpec((256, HALF), lambda j, r: (0, j)),
            pl.BlockSpec((_RB, 16), lambda j, r: (r, 0)),
            pl.BlockSpec((1, 256), lambda j, r: (0, 0)),
        ],
        out_specs=pl.BlockSpec((1, _RB, HALF), lambda j, r: (j, r, 0)),
        out_shape=jax.ShapeDtypeStruct((NC, N, HALF), jnp.float32),
    )(s, W, dinv16, b2d)


def _tc_scale_body(s_ref, dinv_ref, out_ref):
    dinv = dinv_ref[:, 0:1]
    out_ref[0] = dinv * s_ref[0]


def _tc_scale(s, dinv16):
    return pl.pallas_call(
        _tc_scale_body,
        grid=(2, N // _RB),
        in_specs=[
            pl.BlockSpec((1, _RB, HALF), lambda j, r: (j, r, 0)),
            pl.BlockSpec((_RB, 16), lambda j, r: (r, 0)),
        ],
        out_specs=pl.BlockSpec((1, _RB, HALF), lambda j, r: (j, r, 0)),
        out_shape=jax.ShapeDtypeStruct((NC, N, HALF), jnp.float32),
    )(s, dinv16)


def _tc_head_body(sums_ref, cnt_ref, b3_ref, wl_ref, bl_ref, out_ref):
    cnt = cnt_ref[:, 0:1]
    pooled = jnp.concatenate([sums_ref[0], sums_ref[1]], axis=-1)
    pooled = pooled / jnp.maximum(cnt, 1.0)
    pooled = jnp.where(cnt > 0.0, pooled + b3_ref[...], 0.0)
    out_ref[...] = jnp.dot(pooled, wl_ref[...],
                           preferred_element_type=jnp.float32) + bl_ref[...]


def _tc_head(sums, cnt, b3_2d, Wlin, blin2d):
    return pl.pallas_call(
        _tc_head_body,
        out_shape=jax.ShapeDtypeStruct((NUM_GRAPHS, 10), jnp.float32),
    )(sums, cnt, b3_2d, Wlin, blin2d)


@jax.jit
def _gcn(x, edge_index, batch, W1, b1, W2, b2, W3, b3, Wlin, blin):
    ei = edge_index.astype(jnp.int32)
    src2d = ei[0].reshape(ECHUNKS, EK)
    dst2d = ei[1].reshape(ECHUNKS, EK)
    padc = jnp.zeros((CTP * NS - ECHUNKS, EK), jnp.int32)
    src_t = jnp.concatenate([src2d, padc]).reshape(NS, CTP, EK)
    dst_t = jnp.concatenate([dst2d, padc]).reshape(NS, CTP, EK)
    padd = jnp.zeros((DT * NC * NS - ECHUNKS, EK), jnp.int32)
    dstdeg_t = jnp.concatenate([dst2d, padd]).reshape(NC * NS, DT, EK)
    batch3d = batch.astype(jnp.int32).reshape(NCHUNKS, 1, NK)
    ones_in = jnp.ones((EK, HALF), jnp.float32)
    zeros_in = jnp.zeros((RTC, HALF), jnp.float32)
    zeros_pool = jnp.zeros((NUM_GRAPHS, HALF), jnp.float32)

    deg_parts, cnt = _sc_deg_cnt(dstdeg_t, batch3d, ones_in, zeros_in)
    hs1, dinv16 = _tc_l1(x, W1, deg_parts)
    s1 = _sc_conv(hs1, src_t, dst_t)
    hs2 = _tc_layer(s1, W2, b1.reshape(1, 256), dinv16)
    s2 = _sc_conv(hs2, src_t, dst_t)
    hs3 = _tc_layer(s2, W3, b2.reshape(1, 256), dinv16)
    s3 = _sc_conv(hs3, src_t, dst_t)
    h3 = _tc_scale(s3, dinv16)
    sums = _sc_pool(h3, batch3d, zeros_pool)
    return _tc_head(sums, cnt, b3.reshape(1, 256), Wlin, blin.reshape(1, 10))


def kernel(x, edge_index, batch, W1, b1, W2, b2, W3, b3, Wlin, blin):
    return _gcn(x, edge_index, batch, W1, b1, W2, b2, W3, b3, Wlin, blin)


# speedup vs baseline: 1.0862x; 1.0862x over previous
import functools

import jax
import jax.numpy as jnp
from jax import lax
from jax.experimental import pallas as pl
from jax.experimental.pallas import tpu as pltpu
from jax.experimental.pallas import tpu_sc as plsc

N = 10000
E = 320000
NUM_GRAPHS = 64
HALF = 128
NC, NS = 2, 16
EK = 128
ECHUNKS = E // EK
NK = 80
NCHUNKS = N // NK
RT0 = 624
RTC = 640

_mesh = plsc.VectorSubcoreMesh(
    core_axis_name="c", subcore_axis_name="s", num_cores=NC, num_subcores=NS)


DT = 80


@functools.partial(
    pl.kernel,
    out_type=(
        jax.ShapeDtypeStruct((NC, N, HALF), jnp.float32),
        jax.ShapeDtypeStruct((NUM_GRAPHS, HALF), jnp.float32),
    ),
    mesh=_mesh,
    scratch_types=[
        pltpu.VMEM_SHARED((N, HALF), jnp.float32),
        pltpu.VMEM_SHARED((NUM_GRAPHS, HALF), jnp.float32),
        pltpu.VMEM((EK, HALF), jnp.float32),
        pltpu.VMEM((NK, HALF), jnp.float32),
        pltpu.VMEM((DT, EK), jnp.int32),
        pltpu.VMEM((NK,), jnp.int32),
        pltpu.SemaphoreType.DMA,
        pltpu.SemaphoreType.DMA,
    ],
)
def _sc_deg_cnt(dstdeg_t, batch3d, ones_in, zeros_in, deg_out, cnt_out,
                deg_s, cnt_s, ones_e, ones_n, idx_e, idx_n, semA, semB):
    c = lax.axis_index("c")
    s = lax.axis_index("s")
    t = c * NS + s
    pltpu.sync_copy(ones_in, ones_e)
    nch = jnp.clip(ECHUNKS - t * DT, 0, DT)
    pltpu.sync_copy(dstdeg_t.at[t], idx_e)
    pltpu.sync_copy(zeros_in, deg_s.at[pl.ds(s * RT0, RTC)])

    @pl.when((c == 1) & (s == 0))
    def _():
        pltpu.sync_copy(zeros_in.at[pl.ds(0, NUM_GRAPHS)], cnt_s)

    plsc.subcore_barrier()

    def body(m, carry):
        kA = 2 * m
        dA = pltpu.async_copy(ones_e, deg_s.at[idx_e.at[kA]], semA, add=True)
        kB = kA + 1

        @pl.when(kB < nch)
        def _():
            dB = pltpu.async_copy(ones_e, deg_s.at[idx_e.at[kB]], semB,
                                  add=True)
            dA.wait()
            dB.wait()

        @pl.when(kB >= nch)
        def _():
            dA.wait()

        return carry

    lax.fori_loop(0, (nch + 1) // 2, body, 0)

    @pl.when(c == 1)
    def _():
        pltpu.sync_copy(ones_in.at[pl.ds(0, NK)], ones_n)
        nchn = (NCHUNKS - s + NS - 1) // NS

        def bodyn(k, carry):
            i = s + NS * k
            pltpu.sync_copy(batch3d.at[i, 0], idx_n)
            pltpu.sync_copy(ones_n, cnt_s.at[idx_n], add=True)
            return carry

        lax.fori_loop(0, nchn, bodyn, 0)

    plsc.subcore_barrier()
    pltpu.sync_copy(deg_s.at[pl.ds(s * RT0, RTC)],
                    deg_out.at[c].at[pl.ds(s * RT0, RTC)])

    @pl.when((c == 1) & (s == 0))
    def _():
        pltpu.sync_copy(cnt_s, cnt_out)


SEG = 32
NSEG = 5
CTP = SEG * NSEG


@functools.partial(
    pl.kernel,
    out_type=jax.ShapeDtypeStruct((NC, N, HALF), jnp.float32),
    mesh=_mesh,
    scratch_types=[
        pltpu.VMEM_SHARED((N, HALF), jnp.float32),
        pltpu.VMEM((EK, HALF), jnp.float32),
        pltpu.VMEM((EK, HALF), jnp.float32),
        pltpu.VMEM((SEG, EK), jnp.int32),
        pltpu.VMEM((SEG, EK), jnp.int32),
        pltpu.SemaphoreType.DMA,
        pltpu.SemaphoreType.DMA,
    ],
)
def _sc_conv(hs, src_t, dst_t, out, acc, rowsA, rowsB, isrc, idst,
             semA, semB):
    c = lax.axis_index("c")
    s = lax.axis_index("s")
    hs_c = hs.at[c]
    nch = jnp.clip(ECHUNKS - s * CTP, 0, CTP)
    pltpu.sync_copy(hs_c.at[pl.ds(s * RT0, RTC)], acc.at[pl.ds(s * RT0, RTC)])
    plsc.subcore_barrier()

    def seg_body(g, carry0):
        nseg = jnp.clip(nch - g * SEG, 0, SEG)

        @pl.when(nseg > 0)
        def _():
            pltpu.sync_copy(src_t.at[s].at[pl.ds(g * SEG, SEG)], isrc)
            pltpu.sync_copy(dst_t.at[s].at[pl.ds(g * SEG, SEG)], idst)

            def body(m, carry):
                kA = 2 * m
                gA = pltpu.async_copy(hs_c.at[isrc.at[kA]], rowsA, semA)

                @pl.when(kA >= 1)
                def _():
                    pltpu.sync_copy(rowsB, acc.at[idst.at[kA - 1]], add=True)

                gA.wait()
                kB = kA + 1

                @pl.when(kB < nseg)
                def _():
                    gB = pltpu.async_copy(hs_c.at[isrc.at[kB]], rowsB, semB)
                    pltpu.sync_copy(rowsA, acc.at[idst.at[kA]], add=True)
                    gB.wait()

                @pl.when(kB >= nseg)
                def _():
                    pltpu.sync_copy(rowsA, acc.at[idst.at[kA]], add=True)

                return carry

            lax.fori_loop(0, (nseg + 1) // 2, body, 0)

            @pl.when((nseg % 2 == 0) & (nseg >= 2))
            def _():
                pltpu.sync_copy(rowsB, acc.at[idst.at[nseg - 1]], add=True)

        return carry0

    lax.fori_loop(0, NSEG, seg_body, 0)
    plsc.subcore_barrier()
    pltpu.sync_copy(acc.at[pl.ds(s * RT0, RTC)],
                    out.at[c].at[pl.ds(s * RT0, RTC)])


@functools.partial(
    pl.kernel,
    out_type=jax.ShapeDtypeStruct((NC, NUM_GRAPHS, HALF), jnp.float32),
    mesh=_mesh,
    scratch_types=[
        pltpu.VMEM_SHARED((NUM_GRAPHS, HALF), jnp.float32),
        pltpu.VMEM((NK, HALF), jnp.float32),
        pltpu.VMEM((NK,), jnp.int32),
    ],
)
def _sc_pool(h3, batch3d, zeros_in, out, sums_s, rows, idx):
    c = lax.axis_index("c")
    s = lax.axis_index("s")
    h3_c = h3.at[c]

    @pl.when(s == 0)
    def _():
        pltpu.sync_copy(zeros_in, sums_s)

    plsc.subcore_barrier()
    nch = (NCHUNKS - s + NS - 1) // NS

    def body(k, carry):
        i = s + NS * k
        pltpu.sync_copy(batch3d.at[i, 0], idx)
        pltpu.sync_copy(h3_c.at[pl.ds(i * NK, NK)], rows)
        pltpu.sync_copy(rows, sums_s.at[idx], add=True)
        return carry

    lax.fori_loop(0, nch, body, 0)
    plsc.subcore_barrier()

    @pl.when(s == 0)
    def _():
        pltpu.sync_copy(sums_s, out.at[c])


_RB = 1000


def _tc_l1_body(x_ref, w_ref, deg_ref, out_ref, dinv_ref):
    deg = deg_ref[0, :, 0:1] + deg_ref[1, :, 0:1]
    dinv = lax.rsqrt(deg + 1.0)
    dinv_ref[...] = jnp.broadcast_to(dinv, (_RB, 16))
    h = jnp.dot(x_ref[...], w_ref[...], preferred_element_type=jnp.float32)
    out_ref[0] = dinv * h


def _tc_l1(x, W1, deg_parts):
    return pl.pallas_call(
        _tc_l1_body,
        grid=(2, N // _RB),
        in_specs=[
            pl.BlockSpec((_RB, 128), lambda j, r: (r, 0)),
            pl.BlockSpec((128, HALF), lambda j, r: (0, j)),
            pl.BlockSpec((NC, _RB, HALF), lambda j, r: (0, r, 0)),
        ],
        out_specs=[
            pl.BlockSpec((1, _RB, HALF), lambda j, r: (j, r, 0)),
            pl.BlockSpec((_RB, 16), lambda j, r: (r, 0)),
        ],
        out_shape=[
            jax.ShapeDtypeStruct((NC, N, HALF), jnp.float32),
            jax.ShapeDtypeStruct((N, 16), jnp.float32),
        ],
    )(x, W1, deg_parts)


def _tc_layer_body(s_ref, w_ref, dinv_ref, b_ref, out_ref):
    dinv = dinv_ref[:, 0:1]
    agg = jnp.concatenate([s_ref[0], s_ref[1]], axis=-1)
    a = jnp.maximum(dinv * agg + b_ref[...], 0.0)
    h = jnp.dot(a, w_ref[...], preferred_element_type=jnp.float32)
    out_ref[0] = dinv * h


def _tc_layer(s, W, b2d, dinv16):
    return pl.pallas_call(
        _tc_layer_body,
        grid=(2, N // _RB),
        in_specs=[
            pl.BlockSpec((NC, _RB, HALF), lambda j, r: (0, r, 0)),
            pl.BlockSpec((256, HALF), lambda j, r: (0, j)),
            pl.BlockSpec((_RB, 16), lambda j, r: (r, 0)),
            pl.BlockSpec((1, 256), lambda j, r: (0, 0)),
        ],
        out_specs=pl.BlockSpec((1, _RB, HALF), lambda j, r: (j, r, 0)),
        out_shape=jax.ShapeDtypeStruct((NC, N, HALF), jnp.float32),
    )(s, W, dinv16, b2d)


def _tc_scale_body(s_ref, dinv_ref, out_ref):
    dinv = dinv_ref[:, 0:1]
    out_ref[0] = dinv * s_ref[0]


def _tc_scale(s, dinv16):
    return pl.pallas_call(
        _tc_scale_body,
        grid=(2, N // _RB),
        in_specs=[
            pl.BlockSpec((1, _RB, HALF), lambda j, r: (j, r, 0)),
            pl.BlockSpec((_RB, 16), lambda j, r: (r, 0)),
        ],
        out_specs=pl.BlockSpec((1, _RB, HALF), lambda j, r: (j, r, 0)),
        out_shape=jax.ShapeDtypeStruct((NC, N, HALF), jnp.float32),
    )(s, dinv16)


def _tc_head_body(sums_ref, cnt_ref, b3_ref, wl_ref, bl_ref, out_ref):
    cnt = cnt_ref[:, 0:1]
    pooled = jnp.concatenate([sums_ref[0], sums_ref[1]], axis=-1)
    pooled = pooled / jnp.maximum(cnt, 1.0)
    pooled = jnp.where(cnt > 0.0, pooled + b3_ref[...], 0.0)
    out_ref[...] = jnp.dot(pooled, wl_ref[...],
                           preferred_element_type=jnp.float32) + bl_ref[...]


def _tc_head(sums, cnt, b3_2d, Wlin, blin2d):
    return pl.pallas_call(
        _tc_head_body,
        out_shape=jax.ShapeDtypeStruct((NUM_GRAPHS, 10), jnp.float32),
    )(sums, cnt, b3_2d, Wlin, blin2d)


@jax.jit
def _gcn(x, edge_index, batch, W1, b1, W2, b2, W3, b3, Wlin, blin):
    ei = edge_index.astype(jnp.int32)
    src2d = ei[0].reshape(ECHUNKS, EK)
    dst2d = ei[1].reshape(ECHUNKS, EK)
    padc = jnp.zeros((CTP * NS - ECHUNKS, EK), jnp.int32)
    src_t = jnp.concatenate([src2d, padc]).reshape(NS, CTP, EK)
    dst_t = jnp.concatenate([dst2d, padc]).reshape(NS, CTP, EK)
    padd = jnp.zeros((DT * NC * NS - ECHUNKS, EK), jnp.int32)
    dstdeg_t = jnp.concatenate([dst2d, padd]).reshape(NC * NS, DT, EK)
    batch3d = batch.astype(jnp.int32).reshape(NCHUNKS, 1, NK)
    ones_in = jnp.ones((EK, HALF), jnp.float32)
    zeros_in = jnp.zeros((RTC, HALF), jnp.float32)
    zeros_pool = jnp.zeros((NUM_GRAPHS, HALF), jnp.float32)

    deg_parts, cnt = _sc_deg_cnt(dstdeg_t, batch3d, ones_in, zeros_in)
    hs1, dinv16 = _tc_l1(x, W1, deg_parts)
    s1 = _sc_conv(hs1, src_t, dst_t)
    hs2 = _tc_layer(s1, W2, b1.reshape(1, 256), dinv16)
    s2 = _sc_conv(hs2, src_t, dst_t)
    hs3 = _tc_layer(s2, W3, b2.reshape(1, 256), dinv16)
    s3 = _sc_conv(hs3, src_t, dst_t)
    h3 = _tc_scale(s3, dinv16)
    sums = _sc_pool(h3, batch3d, zeros_pool)
    return _tc_head(sums, cnt, b3.reshape(1, 256), Wlin, blin.reshape(1, 10))


def kernel(x, edge_index, batch, W1, b1, W2, b2, W3, b3, Wlin, blin):
    return _gcn(x, edge_index, batch, W1, b1, W2, b2, W3, b3, Wlin, blin)


# speedup vs baseline: 1.1045x; 1.0169x over previous
import functools

import jax
import jax.numpy as jnp
from jax import lax
from jax.experimental import pallas as pl
from jax.experimental.pallas import tpu as pltpu
from jax.experimental.pallas import tpu_sc as plsc

N = 10000
E = 320000
NUM_GRAPHS = 64
HALF = 128
NC, NS = 2, 16
EK = 128
ECHUNKS = E // EK
NK = 80
NCHUNKS = N // NK
RT0 = 624
RTC = 640

_mesh = plsc.VectorSubcoreMesh(
    core_axis_name="c", subcore_axis_name="s", num_cores=NC, num_subcores=NS)


DT = 80


@functools.partial(
    pl.kernel,
    out_type=(
        jax.ShapeDtypeStruct((NC, N, HALF), jnp.float32),
        jax.ShapeDtypeStruct((NUM_GRAPHS, HALF), jnp.float32),
    ),
    mesh=_mesh,
    scratch_types=[
        pltpu.VMEM_SHARED((N, HALF), jnp.float32),
        pltpu.VMEM_SHARED((NUM_GRAPHS, HALF), jnp.float32),
        pltpu.VMEM((EK, HALF), jnp.float32),
        pltpu.VMEM((NK, HALF), jnp.float32),
        pltpu.VMEM((DT, EK), jnp.int32),
        pltpu.VMEM((NK,), jnp.int32),
        pltpu.SemaphoreType.DMA,
        pltpu.SemaphoreType.DMA,
    ],
)
def _sc_deg_cnt(dstdeg_t, batch3d, ones_in, zeros_in, deg_out, cnt_out,
                deg_s, cnt_s, ones_e, ones_n, idx_e, idx_n, semA, semB):
    c = lax.axis_index("c")
    s = lax.axis_index("s")
    t = c * NS + s
    pltpu.sync_copy(ones_in, ones_e)
    nch = jnp.clip(ECHUNKS - t * DT, 0, DT)
    pltpu.sync_copy(dstdeg_t.at[t], idx_e)
    pltpu.sync_copy(zeros_in, deg_s.at[pl.ds(s * RT0, RTC)])

    @pl.when((c == 1) & (s == 0))
    def _():
        pltpu.sync_copy(zeros_in.at[pl.ds(0, NUM_GRAPHS)], cnt_s)

    plsc.subcore_barrier()

    def body(m, carry):
        kA = 2 * m
        dA = pltpu.async_copy(ones_e, deg_s.at[idx_e.at[kA]], semA, add=True)
        kB = kA + 1

        @pl.when(kB < nch)
        def _():
            dB = pltpu.async_copy(ones_e, deg_s.at[idx_e.at[kB]], semB,
                                  add=True)
            dA.wait()
            dB.wait()

        @pl.when(kB >= nch)
        def _():
            dA.wait()

        return carry

    lax.fori_loop(0, (nch + 1) // 2, body, 0)

    @pl.when(c == 1)
    def _():
        pltpu.sync_copy(ones_in.at[pl.ds(0, NK)], ones_n)
        nchn = (NCHUNKS - s + NS - 1) // NS

        def bodyn(k, carry):
            i = s + NS * k
            pltpu.sync_copy(batch3d.at[i, 0], idx_n)
            pltpu.sync_copy(ones_n, cnt_s.at[idx_n], add=True)
            return carry

        lax.fori_loop(0, nchn, bodyn, 0)

    plsc.subcore_barrier()
    pltpu.sync_copy(deg_s.at[pl.ds(s * RT0, RTC)],
                    deg_out.at[c].at[pl.ds(s * RT0, RTC)])

    @pl.when((c == 1) & (s == 0))
    def _():
        pltpu.sync_copy(cnt_s, cnt_out)


CT = 157
SEG = 32
NSEG = 5
CTP = SEG * NSEG


@functools.partial(
    pl.kernel,
    out_type=jax.ShapeDtypeStruct((NC, N, HALF), jnp.float32),
    mesh=_mesh,
    scratch_types=[
        pltpu.VMEM_SHARED((N, HALF), jnp.float32),
        pltpu.VMEM((EK, HALF), jnp.float32),
        pltpu.VMEM((EK, HALF), jnp.float32),
        pltpu.VMEM((SEG, EK), jnp.int32),
        pltpu.VMEM((SEG, EK), jnp.int32),
        pltpu.SemaphoreType.DMA,
        pltpu.SemaphoreType.DMA,
    ],
)
def _sc_conv(hs, src_t, dst_t, out, acc, rowsA, rowsB, isrc, idst,
             semA, semB):
    c = lax.axis_index("c")
    s = lax.axis_index("s")
    hs_c = hs.at[c]
    nch = jnp.minimum(CT, ECHUNKS - s * CT)
    pltpu.sync_copy(hs_c.at[pl.ds(s * RT0, RTC)], acc.at[pl.ds(s * RT0, RTC)])
    plsc.subcore_barrier()

    def seg_body(g, carry0):
        nseg = jnp.clip(nch - g * SEG, 0, SEG)

        @pl.when(nseg > 0)
        def _():
            pltpu.sync_copy(src_t.at[s].at[pl.ds(g * SEG, SEG)], isrc)
            pltpu.sync_copy(dst_t.at[s].at[pl.ds(g * SEG, SEG)], idst)

            def body(m, carry):
                kA = 2 * m
                gA = pltpu.async_copy(hs_c.at[isrc.at[kA]], rowsA, semA)

                @pl.when(kA >= 1)
                def _():
                    pltpu.sync_copy(rowsB, acc.at[idst.at[kA - 1]], add=True)

                gA.wait()
                kB = kA + 1

                @pl.when(kB < nseg)
                def _():
                    gB = pltpu.async_copy(hs_c.at[isrc.at[kB]], rowsB, semB)
                    pltpu.sync_copy(rowsA, acc.at[idst.at[kA]], add=True)
                    gB.wait()

                @pl.when(kB >= nseg)
                def _():
                    pltpu.sync_copy(rowsA, acc.at[idst.at[kA]], add=True)

                return carry

            lax.fori_loop(0, (nseg + 1) // 2, body, 0)

            @pl.when((nseg % 2 == 0) & (nseg >= 2))
            def _():
                pltpu.sync_copy(rowsB, acc.at[idst.at[nseg - 1]], add=True)

        return carry0

    lax.fori_loop(0, NSEG, seg_body, 0)
    plsc.subcore_barrier()
    pltpu.sync_copy(acc.at[pl.ds(s * RT0, RTC)],
                    out.at[c].at[pl.ds(s * RT0, RTC)])


@functools.partial(
    pl.kernel,
    out_type=jax.ShapeDtypeStruct((NC, NUM_GRAPHS, HALF), jnp.float32),
    mesh=_mesh,
    scratch_types=[
        pltpu.VMEM_SHARED((NUM_GRAPHS, HALF), jnp.float32),
        pltpu.VMEM((NK, HALF), jnp.float32),
        pltpu.VMEM((NK,), jnp.int32),
    ],
)
def _sc_pool(h3, batch3d, zeros_in, out, sums_s, rows, idx):
    c = lax.axis_index("c")
    s = lax.axis_index("s")
    h3_c = h3.at[c]

    @pl.when(s == 0)
    def _():
        pltpu.sync_copy(zeros_in, sums_s)

    plsc.subcore_barrier()
    nch = (NCHUNKS - s + NS - 1) // NS

    def body(k, carry):
        i = s + NS * k
        pltpu.sync_copy(batch3d.at[i, 0], idx)
        pltpu.sync_copy(h3_c.at[pl.ds(i * NK, NK)], rows)
        pltpu.sync_copy(rows, sums_s.at[idx], add=True)
        return carry

    lax.fori_loop(0, nch, body, 0)
    plsc.subcore_barrier()

    @pl.when(s == 0)
    def _():
        pltpu.sync_copy(sums_s, out.at[c])


_RB = 1000


def _tc_l1_body(x_ref, w_ref, deg_ref, out_ref, dinv_ref):
    deg = deg_ref[0, :, 0:1] + deg_ref[1, :, 0:1]
    dinv = lax.rsqrt(deg + 1.0)
    dinv_ref[...] = jnp.broadcast_to(dinv, (_RB, 16))
    h = jnp.dot(x_ref[...], w_ref[...], preferred_element_type=jnp.float32)
    out_ref[0] = dinv * h


def _tc_l1(x, W1, deg_parts):
    return pl.pallas_call(
        _tc_l1_body,
        grid=(2, N // _RB),
        in_specs=[
            pl.BlockSpec((_RB, 128), lambda j, r: (r, 0)),
            pl.BlockSpec((128, HALF), lambda j, r: (0, j)),
            pl.BlockSpec((NC, _RB, HALF), lambda j, r: (0, r, 0)),
        ],
        out_specs=[
            pl.BlockSpec((1, _RB, HALF), lambda j, r: (j, r, 0)),
            pl.BlockSpec((_RB, 16), lambda j, r: (r, 0)),
        ],
        out_shape=[
            jax.ShapeDtypeStruct((NC, N, HALF), jnp.float32),
            jax.ShapeDtypeStruct((N, 16), jnp.float32),
        ],
    )(x, W1, deg_parts)


def _tc_layer_body(s_ref, w_ref, dinv_ref, b_ref, out_ref):
    dinv = dinv_ref[:, 0:1]
    agg = jnp.concatenate([s_ref[0], s_ref[1]], axis=-1)
    a = jnp.maximum(dinv * agg + b_ref[...], 0.0)
    h = jnp.dot(a, w_ref[...], preferred_element_type=jnp.float32)
    out_ref[0] = dinv * h


def _tc_layer(s, W, b2d, dinv16):
    return pl.pallas_call(
        _tc_layer_body,
        grid=(2, N // _RB),
        in_specs=[
            pl.BlockSpec((NC, _RB, HALF), lambda j, r: (0, r, 0)),
            pl.BlockSpec((256, HALF), lambda j, r: (0, j)),
            pl.BlockSpec((_RB, 16), lambda j, r: (r, 0)),
            pl.BlockSpec((1, 256), lambda j, r: (0, 0)),
        ],
        out_specs=pl.BlockSpec((1, _RB, HALF), lambda j, r: (j, r, 0)),
        out_shape=jax.ShapeDtypeStruct((NC, N, HALF), jnp.float32),
    )(s, W, dinv16, b2d)


def _tc_scale_body(s_ref, dinv_ref, out_ref):
    dinv = dinv_ref[:, 0:1]
    out_ref[0] = dinv * s_ref[0]


def _tc_scale(s, dinv16):
    return pl.pallas_call(
        _tc_scale_body,
        grid=(2, N // _RB),
        in_specs=[
            pl.BlockSpec((1, _RB, HALF), lambda j, r: (j, r, 0)),
            pl.BlockSpec((_RB, 16), lambda j, r: (r, 0)),
        ],
        out_specs=pl.BlockSpec((1, _RB, HALF), lambda j, r: (j, r, 0)),
        out_shape=jax.ShapeDtypeStruct((NC, N, HALF), jnp.float32),
    )(s, dinv16)


def _tc_head_body(sums_ref, cnt_ref, b3_ref, wl_ref, bl_ref, out_ref):
    cnt = cnt_ref[:, 0:1]
    pooled = jnp.concatenate([sums_ref[0], sums_ref[1]], axis=-1)
    pooled = pooled / jnp.maximum(cnt, 1.0)
    pooled = jnp.where(cnt > 0.0, pooled + b3_ref[...], 0.0)
    out_ref[...] = jnp.dot(pooled, wl_ref[...],
                           preferred_element_type=jnp.float32) + bl_ref[...]


def _tc_head(sums, cnt, b3_2d, Wlin, blin2d):
    return pl.pallas_call(
        _tc_head_body,
        out_shape=jax.ShapeDtypeStruct((NUM_GRAPHS, 10), jnp.float32),
    )(sums, cnt, b3_2d, Wlin, blin2d)


@jax.jit
def _gcn(x, edge_index, batch, W1, b1, W2, b2, W3, b3, Wlin, blin):
    ei = edge_index.astype(jnp.int32)
    src2d = ei[0].reshape(ECHUNKS, EK)
    dst2d = ei[1].reshape(ECHUNKS, EK)
    tiler = jnp.clip(jnp.arange(NS)[:, None] * CT + jnp.arange(CTP)[None, :],
                     0, ECHUNKS - 1).reshape(-1)
    src_t = src2d[tiler].reshape(NS, CTP, EK)
    dst_t = dst2d[tiler].reshape(NS, CTP, EK)
    padd = jnp.zeros((DT * NC * NS - ECHUNKS, EK), jnp.int32)
    dstdeg_t = jnp.concatenate([dst2d, padd]).reshape(NC * NS, DT, EK)
    batch3d = batch.astype(jnp.int32).reshape(NCHUNKS, 1, NK)
    ones_in = jnp.ones((EK, HALF), jnp.float32)
    zeros_in = jnp.zeros((RTC, HALF), jnp.float32)
    zeros_pool = jnp.zeros((NUM_GRAPHS, HALF), jnp.float32)

    deg_parts, cnt = _sc_deg_cnt(dstdeg_t, batch3d, ones_in, zeros_in)
    hs1, dinv16 = _tc_l1(x, W1, deg_parts)
    s1 = _sc_conv(hs1, src_t, dst_t)
    hs2 = _tc_layer(s1, W2, b1.reshape(1, 256), dinv16)
    s2 = _sc_conv(hs2, src_t, dst_t)
    hs3 = _tc_layer(s2, W3, b2.reshape(1, 256), dinv16)
    s3 = _sc_conv(hs3, src_t, dst_t)
    h3 = _tc_scale(s3, dinv16)
    sums = _sc_pool(h3, batch3d, zeros_pool)
    return _tc_head(sums, cnt, b3.reshape(1, 256), Wlin, blin.reshape(1, 10))


def kernel(x, edge_index, batch, W1, b1, W2, b2, W3, b3, Wlin, blin):
    return _gcn(x, edge_index, batch, W1, b1, W2, b2, W3, b3, Wlin, blin)


# speedup vs baseline: 1.1247x; 1.0182x over previous
import functools

import jax
import jax.numpy as jnp
from jax import lax
from jax.experimental import pallas as pl
from jax.experimental.pallas import tpu as pltpu
from jax.experimental.pallas import tpu_sc as plsc

N = 10000
E = 320000
NUM_GRAPHS = 64
HALF = 128
NC, NS = 2, 16
EK = 128
ECHUNKS = E // EK
NK = 80
NCHUNKS = N // NK
RT0 = 624
RTC = 640

_mesh = plsc.VectorSubcoreMesh(
    core_axis_name="c", subcore_axis_name="s", num_cores=NC, num_subcores=NS)


DT = 80


@functools.partial(
    pl.kernel,
    out_type=(
        jax.ShapeDtypeStruct((NC, N, HALF), jnp.float32),
        jax.ShapeDtypeStruct((NUM_GRAPHS, HALF), jnp.float32),
    ),
    mesh=_mesh,
    scratch_types=[
        pltpu.VMEM_SHARED((N, HALF), jnp.float32),
        pltpu.VMEM_SHARED((NUM_GRAPHS, HALF), jnp.float32),
        pltpu.VMEM((EK, HALF), jnp.float32),
        pltpu.VMEM((NK, HALF), jnp.float32),
        pltpu.VMEM((DT, EK), jnp.int32),
        pltpu.VMEM((NK,), jnp.int32),
        pltpu.SemaphoreType.DMA,
        pltpu.SemaphoreType.DMA,
    ],
)
def _sc_deg_cnt(dstdeg_t, batch3d, ones_in, zeros_in, deg_out, cnt_out,
                deg_s, cnt_s, ones_e, ones_n, idx_e, idx_n, semA, semB):
    c = lax.axis_index("c")
    s = lax.axis_index("s")
    t = c * NS + s
    pltpu.sync_copy(ones_in, ones_e)
    nch = jnp.clip(ECHUNKS - t * DT, 0, DT)
    pltpu.sync_copy(dstdeg_t.at[t], idx_e)
    pltpu.sync_copy(zeros_in, deg_s.at[pl.ds(s * RT0, RTC)])

    @pl.when((c == 1) & (s == 0))
    def _():
        pltpu.sync_copy(zeros_in.at[pl.ds(0, NUM_GRAPHS)], cnt_s)

    plsc.subcore_barrier()

    def body(m, carry):
        kA = 2 * m
        dA = pltpu.async_copy(ones_e, deg_s.at[idx_e.at[kA]], semA, add=True)
        kB = kA + 1

        @pl.when(kB < nch)
        def _():
            dB = pltpu.async_copy(ones_e, deg_s.at[idx_e.at[kB]], semB,
                                  add=True)
            dA.wait()
            dB.wait()

        @pl.when(kB >= nch)
        def _():
            dA.wait()

        return carry

    lax.fori_loop(0, (nch + 1) // 2, body, 0)

    @pl.when(c == 1)
    def _():
        pltpu.sync_copy(ones_in.at[pl.ds(0, NK)], ones_n)
        nchn = (NCHUNKS - s + NS - 1) // NS

        def bodyn(k, carry):
            i = s + NS * k
            pltpu.sync_copy(batch3d.at[i, 0], idx_n)
            pltpu.sync_copy(ones_n, cnt_s.at[idx_n], add=True)
            return carry

        lax.fori_loop(0, nchn, bodyn, 0)

    plsc.subcore_barrier()
    pltpu.sync_copy(deg_s.at[pl.ds(s * RT0, RTC)],
                    deg_out.at[c].at[pl.ds(s * RT0, RTC)])

    @pl.when((c == 1) & (s == 0))
    def _():
        pltpu.sync_copy(cnt_s, cnt_out)


CT = 157
SEG = 32
NSEG = 5
CTP = SEG * NSEG


@functools.partial(
    pl.kernel,
    out_type=jax.ShapeDtypeStruct((NC, N, HALF), jnp.float32),
    mesh=_mesh,
    scratch_types=[
        pltpu.VMEM_SHARED((N, HALF), jnp.float32),
        pltpu.VMEM((EK, HALF), jnp.float32),
        pltpu.VMEM((EK, HALF), jnp.float32),
        pltpu.VMEM((SEG, EK), jnp.int32),
        pltpu.VMEM((SEG, EK), jnp.int32),
        pltpu.SemaphoreType.DMA,
        pltpu.SemaphoreType.DMA,
    ],
)
def _sc_conv(hs, src_t, dst_t, out, acc, rowsA, rowsB, isrc, idst,
             semA, semB):
    c = lax.axis_index("c")
    s = lax.axis_index("s")
    hs_c = hs.at[c]
    nch = jnp.minimum(CT, ECHUNKS - s * CT)
    pltpu.sync_copy(hs_c.at[pl.ds(s * RT0, RTC)], acc.at[pl.ds(s * RT0, RTC)])
    plsc.subcore_barrier()

    def seg_body(g, carry0):
        nseg = jnp.clip(nch - g * SEG, 0, SEG)

        @pl.when(nseg > 0)
        def _():
            pltpu.sync_copy(src_t.at[s].at[pl.ds(g * SEG, SEG)], isrc)
            pltpu.sync_copy(dst_t.at[s].at[pl.ds(g * SEG, SEG)], idst)

            def body(m, carry):
                kA = 2 * m
                gA = pltpu.async_copy(hs_c.at[isrc.at[kA]], rowsA, semA)

                @pl.when(kA >= 1)
                def _():
                    pltpu.sync_copy(rowsB, acc.at[idst.at[kA - 1]], add=True)

                gA.wait()
                kB = kA + 1

                @pl.when(kB < nseg)
                def _():
                    gB = pltpu.async_copy(hs_c.at[isrc.at[kB]], rowsB, semB)
                    pltpu.sync_copy(rowsA, acc.at[idst.at[kA]], add=True)
                    gB.wait()

                @pl.when(kB >= nseg)
                def _():
                    pltpu.sync_copy(rowsA, acc.at[idst.at[kA]], add=True)

                return carry

            lax.fori_loop(0, (nseg + 1) // 2, body, 0)

            @pl.when((nseg % 2 == 0) & (nseg >= 2))
            def _():
                pltpu.sync_copy(rowsB, acc.at[idst.at[nseg - 1]], add=True)

        return carry0

    lax.fori_loop(0, NSEG, seg_body, 0)
    plsc.subcore_barrier()
    pltpu.sync_copy(acc.at[pl.ds(s * RT0, RTC)],
                    out.at[c].at[pl.ds(s * RT0, RTC)])


@functools.partial(
    pl.kernel,
    out_type=jax.ShapeDtypeStruct((NC, NUM_GRAPHS, HALF), jnp.float32),
    mesh=_mesh,
    scratch_types=[
        pltpu.VMEM_SHARED((NUM_GRAPHS, HALF), jnp.float32),
        pltpu.VMEM((NK, HALF), jnp.float32),
        pltpu.VMEM((NK,), jnp.int32),
    ],
)
def _sc_pool(h3, batch3d, zeros_in, out, sums_s, rows, idx):
    c = lax.axis_index("c")
    s = lax.axis_index("s")
    h3_c = h3.at[c]

    @pl.when(s == 0)
    def _():
        pltpu.sync_copy(zeros_in, sums_s)

    plsc.subcore_barrier()
    nch = (NCHUNKS - s + NS - 1) // NS

    def body(k, carry):
        i = s + NS * k
        pltpu.sync_copy(batch3d.at[i, 0], idx)
        pltpu.sync_copy(h3_c.at[pl.ds(i * NK, NK)], rows)
        pltpu.sync_copy(rows, sums_s.at[idx], add=True)
        return carry

    lax.fori_loop(0, nch, body, 0)
    plsc.subcore_barrier()

    @pl.when(s == 0)
    def _():
        pltpu.sync_copy(sums_s, out.at[c])


_RB = 2000


def _tc_l1_body(x_ref, w_ref, deg_ref, out_ref, dinv_ref):
    deg = deg_ref[0, :, 0:1] + deg_ref[1, :, 0:1]
    dinv = lax.rsqrt(deg + 1.0)
    dinv_ref[...] = jnp.broadcast_to(dinv, (_RB, 16))
    h = jnp.dot(x_ref[...], w_ref[...], preferred_element_type=jnp.float32)
    out_ref[0] = dinv * h


def _tc_l1(x, W1, deg_parts):
    return pl.pallas_call(
        _tc_l1_body,
        grid=(2, N // _RB),
        in_specs=[
            pl.BlockSpec((_RB, 128), lambda j, r: (r, 0)),
            pl.BlockSpec((128, HALF), lambda j, r: (0, j)),
            pl.BlockSpec((NC, _RB, HALF), lambda j, r: (0, r, 0)),
        ],
        out_specs=[
            pl.BlockSpec((1, _RB, HALF), lambda j, r: (j, r, 0)),
            pl.BlockSpec((_RB, 16), lambda j, r: (r, 0)),
        ],
        out_shape=[
            jax.ShapeDtypeStruct((NC, N, HALF), jnp.float32),
            jax.ShapeDtypeStruct((N, 16), jnp.float32),
        ],
    )(x, W1, deg_parts)


def _tc_layer_body(s_ref, w_ref, dinv_ref, b_ref, out_ref):
    dinv = dinv_ref[:, 0:1]
    agg = jnp.concatenate([s_ref[0], s_ref[1]], axis=-1)
    a = jnp.maximum(dinv * agg + b_ref[...], 0.0)
    h = jnp.dot(a, w_ref[...], preferred_element_type=jnp.float32)
    out_ref[0] = dinv * h


def _tc_layer(s, W, b2d, dinv16):
    return pl.pallas_call(
        _tc_layer_body,
        grid=(2, N // _RB),
        in_specs=[
            pl.BlockSpec((NC, _RB, HALF), lambda j, r: (0, r, 0)),
            pl.BlockSpec((256, HALF), lambda j, r: (0, j)),
            pl.BlockSpec((_RB, 16), lambda j, r: (r, 0)),
            pl.BlockSpec((1, 256), lambda j, r: (0, 0)),
        ],
        out_specs=pl.BlockSpec((1, _RB, HALF), lambda j, r: (j, r, 0)),
        out_shape=jax.ShapeDtypeStruct((NC, N, HALF), jnp.float32),
    )(s, W, dinv16, b2d)


def _tc_scale_body(s_ref, dinv_ref, out_ref):
    dinv = dinv_ref[:, 0:1]
    out_ref[0] = dinv * s_ref[0]


def _tc_scale(s, dinv16):
    return pl.pallas_call(
        _tc_scale_body,
        grid=(2, N // _RB),
        in_specs=[
            pl.BlockSpec((1, _RB, HALF), lambda j, r: (j, r, 0)),
            pl.BlockSpec((_RB, 16), lambda j, r: (r, 0)),
        ],
        out_specs=pl.BlockSpec((1, _RB, HALF), lambda j, r: (j, r, 0)),
        out_shape=jax.ShapeDtypeStruct((NC, N, HALF), jnp.float32),
    )(s, dinv16)


def _tc_head_body(sums_ref, cnt_ref, b3_ref, wl_ref, bl_ref, out_ref):
    cnt = cnt_ref[:, 0:1]
    pooled = jnp.concatenate([sums_ref[0], sums_ref[1]], axis=-1)
    pooled = pooled / jnp.maximum(cnt, 1.0)
    pooled = jnp.where(cnt > 0.0, pooled + b3_ref[...], 0.0)
    out_ref[...] = jnp.dot(pooled, wl_ref[...],
                           preferred_element_type=jnp.float32) + bl_ref[...]


def _tc_head(sums, cnt, b3_2d, Wlin, blin2d):
    return pl.pallas_call(
        _tc_head_body,
        out_shape=jax.ShapeDtypeStruct((NUM_GRAPHS, 10), jnp.float32),
    )(sums, cnt, b3_2d, Wlin, blin2d)


@jax.jit
def _gcn(x, edge_index, batch, W1, b1, W2, b2, W3, b3, Wlin, blin):
    ei = edge_index.astype(jnp.int32)
    src2d = ei[0].reshape(ECHUNKS, EK)
    dst2d = ei[1].reshape(ECHUNKS, EK)
    tiler = jnp.clip(jnp.arange(NS)[:, None] * CT + jnp.arange(CTP)[None, :],
                     0, ECHUNKS - 1).reshape(-1)
    src_t = src2d[tiler].reshape(NS, CTP, EK)
    dst_t = dst2d[tiler].reshape(NS, CTP, EK)
    padd = jnp.zeros((DT * NC * NS - ECHUNKS, EK), jnp.int32)
    dstdeg_t = jnp.concatenate([dst2d, padd]).reshape(NC * NS, DT, EK)
    batch3d = batch.astype(jnp.int32).reshape(NCHUNKS, 1, NK)
    ones_in = jnp.ones((EK, HALF), jnp.float32)
    zeros_in = jnp.zeros((RTC, HALF), jnp.float32)
    zeros_pool = jnp.zeros((NUM_GRAPHS, HALF), jnp.float32)

    deg_parts, cnt = _sc_deg_cnt(dstdeg_t, batch3d, ones_in, zeros_in)
    hs1, dinv16 = _tc_l1(x, W1, deg_parts)
    s1 = _sc_conv(hs1, src_t, dst_t)
    hs2 = _tc_layer(s1, W2, b1.reshape(1, 256), dinv16)
    s2 = _sc_conv(hs2, src_t, dst_t)
    hs3 = _tc_layer(s2, W3, b2.reshape(1, 256), dinv16)
    s3 = _sc_conv(hs3, src_t, dst_t)
    h3 = _tc_scale(s3, dinv16)
    sums = _sc_pool(h3, batch3d, zeros_pool)
    return _tc_head(sums, cnt, b3.reshape(1, 256), Wlin, blin.reshape(1, 10))


def kernel(x, edge_index, batch, W1, b1, W2, b2, W3, b3, Wlin, blin):
    return _gcn(x, edge_index, batch, W1, b1, W2, b2, W3, b3, Wlin, blin)


# speedup vs baseline: 1.1326x; 1.0070x over previous
import functools

import jax
import jax.numpy as jnp
from jax import lax
from jax.experimental import pallas as pl
from jax.experimental.pallas import tpu as pltpu
from jax.experimental.pallas import tpu_sc as plsc

N = 10000
E = 320000
NUM_GRAPHS = 64
HALF = 128
NC, NS = 2, 16
EK = 128
ECHUNKS = E // EK
NK = 80
NCHUNKS = N // NK
RT0 = 624
RTC = 640

_mesh = plsc.VectorSubcoreMesh(
    core_axis_name="c", subcore_axis_name="s", num_cores=NC, num_subcores=NS)


DT = 80


@functools.partial(
    pl.kernel,
    out_type=(
        jax.ShapeDtypeStruct((NC, N, HALF), jnp.float32),
        jax.ShapeDtypeStruct((NUM_GRAPHS, HALF), jnp.float32),
    ),
    mesh=_mesh,
    scratch_types=[
        pltpu.VMEM_SHARED((N, HALF), jnp.float32),
        pltpu.VMEM_SHARED((NUM_GRAPHS, HALF), jnp.float32),
        pltpu.VMEM((EK, HALF), jnp.float32),
        pltpu.VMEM((NK, HALF), jnp.float32),
        pltpu.VMEM((DT, EK), jnp.int32),
        pltpu.VMEM((NK,), jnp.int32),
        pltpu.SemaphoreType.DMA,
        pltpu.SemaphoreType.DMA,
    ],
)
def _sc_deg_cnt(dstdeg_t, batch3d, ones_in, zeros_in, deg_out, cnt_out,
                deg_s, cnt_s, ones_e, ones_n, idx_e, idx_n, semA, semB):
    c = lax.axis_index("c")
    s = lax.axis_index("s")
    t = c * NS + s
    pltpu.sync_copy(ones_in, ones_e)
    nch = jnp.clip(ECHUNKS - t * DT, 0, DT)
    pltpu.sync_copy(dstdeg_t.at[t], idx_e)
    pltpu.sync_copy(zeros_in, deg_s.at[pl.ds(s * RT0, RTC)])

    @pl.when((c == 1) & (s == 0))
    def _():
        pltpu.sync_copy(zeros_in.at[pl.ds(0, NUM_GRAPHS)], cnt_s)

    plsc.subcore_barrier()

    def body(m, carry):
        kA = 2 * m
        dA = pltpu.async_copy(ones_e, deg_s.at[idx_e.at[kA]], semA, add=True)
        kB = kA + 1

        @pl.when(kB < nch)
        def _():
            dB = pltpu.async_copy(ones_e, deg_s.at[idx_e.at[kB]], semB,
                                  add=True)
            dA.wait()
            dB.wait()

        @pl.when(kB >= nch)
        def _():
            dA.wait()

        return carry

    lax.fori_loop(0, (nch + 1) // 2, body, 0)

    @pl.when(c == 1)
    def _():
        pltpu.sync_copy(ones_in.at[pl.ds(0, NK)], ones_n)
        nchn = (NCHUNKS - s + NS - 1) // NS

        def bodyn(k, carry):
            i = s + NS * k
            pltpu.sync_copy(batch3d.at[i, 0], idx_n)
            pltpu.sync_copy(ones_n, cnt_s.at[idx_n], add=True)
            return carry

        lax.fori_loop(0, nchn, bodyn, 0)

    plsc.subcore_barrier()
    pltpu.sync_copy(deg_s.at[pl.ds(s * RT0, RTC)],
                    deg_out.at[c].at[pl.ds(s * RT0, RTC)])

    @pl.when((c == 1) & (s == 0))
    def _():
        pltpu.sync_copy(cnt_s, cnt_out)


CT = 157
SEG = 32
NSEG = 5
CTP = SEG * NSEG


@functools.partial(
    pl.kernel,
    out_type=jax.ShapeDtypeStruct((NC, N, HALF), jnp.float32),
    mesh=_mesh,
    scratch_types=[
        pltpu.VMEM_SHARED((N, HALF), jnp.float32),
        pltpu.VMEM((EK, HALF), jnp.float32),
        pltpu.VMEM((EK, HALF), jnp.float32),
        pltpu.VMEM((SEG, EK), jnp.int32),
        pltpu.VMEM((SEG, EK), jnp.int32),
        pltpu.SemaphoreType.DMA,
        pltpu.SemaphoreType.DMA,
    ],
)
def _sc_conv(hs, src_t, dst_t, out, acc, rowsA, rowsB, isrc, idst,
             semA, semB):
    c = lax.axis_index("c")
    s = lax.axis_index("s")
    hs_c = hs.at[c]
    nch = jnp.minimum(CT, ECHUNKS - s * CT)
    pltpu.sync_copy(hs_c.at[pl.ds(s * RT0, RTC)], acc.at[pl.ds(s * RT0, RTC)])
    plsc.subcore_barrier()

    def seg_body(g, carry0):
        nseg = jnp.clip(nch - g * SEG, 0, SEG)

        @pl.when(nseg > 0)
        def _():
            pltpu.sync_copy(src_t.at[s].at[pl.ds(g * SEG, SEG)], isrc)
            pltpu.sync_copy(dst_t.at[s].at[pl.ds(g * SEG, SEG)], idst)

            def body(m, carry):
                kA = 2 * m
                gA = pltpu.async_copy(hs_c.at[isrc.at[kA]], rowsA, semA)

                @pl.when(kA >= 1)
                def _():
                    pltpu.sync_copy(rowsB, acc.at[idst.at[kA - 1]], add=True)

                gA.wait()
                kB = kA + 1

                @pl.when(kB < nseg)
                def _():
                    gB = pltpu.async_copy(hs_c.at[isrc.at[kB]], rowsB, semB)
                    pltpu.sync_copy(rowsA, acc.at[idst.at[kA]], add=True)
                    gB.wait()

                @pl.when(kB >= nseg)
                def _():
                    pltpu.sync_copy(rowsA, acc.at[idst.at[kA]], add=True)

                return carry

            lax.fori_loop(0, (nseg + 1) // 2, body, 0)

            @pl.when((nseg % 2 == 0) & (nseg >= 2))
            def _():
                pltpu.sync_copy(rowsB, acc.at[idst.at[nseg - 1]], add=True)

        return carry0

    lax.fori_loop(0, NSEG, seg_body, 0)
    plsc.subcore_barrier()
    pltpu.sync_copy(acc.at[pl.ds(s * RT0, RTC)],
                    out.at[c].at[pl.ds(s * RT0, RTC)])


@functools.partial(
    pl.kernel,
    out_type=jax.ShapeDtypeStruct((NC, NUM_GRAPHS, HALF), jnp.float32),
    mesh=_mesh,
    scratch_types=[
        pltpu.VMEM_SHARED((NUM_GRAPHS, HALF), jnp.float32),
        pltpu.VMEM((NK, HALF), jnp.float32),
        pltpu.VMEM((NK,), jnp.int32),
    ],
)
def _sc_pool(h3, batch3d, zeros_in, out, sums_s, rows, idx):
    c = lax.axis_index("c")
    s = lax.axis_index("s")
    h3_c = h3.at[c]

    @pl.when(s == 0)
    def _():
        pltpu.sync_copy(zeros_in, sums_s)

    plsc.subcore_barrier()
    nch = (NCHUNKS - s + NS - 1) // NS

    def body(k, carry):
        i = s + NS * k
        pltpu.sync_copy(batch3d.at[i, 0], idx)
        pltpu.sync_copy(h3_c.at[pl.ds(i * NK, NK)], rows)
        pltpu.sync_copy(rows, sums_s.at[idx], add=True)
        return carry

    lax.fori_loop(0, nch, body, 0)
    plsc.subcore_barrier()

    @pl.when(s == 0)
    def _():
        pltpu.sync_copy(sums_s, out.at[c])


_RB = 5000


def _tc_l1_body(x_ref, w_ref, deg_ref, out_ref, dinv_ref):
    deg = deg_ref[0, :, 0:1] + deg_ref[1, :, 0:1]
    dinv = lax.rsqrt(deg + 1.0)
    dinv_ref[...] = jnp.broadcast_to(dinv, (_RB, 16))
    h = jnp.dot(x_ref[...], w_ref[...], preferred_element_type=jnp.float32)
    out_ref[0] = dinv * h


def _tc_l1(x, W1, deg_parts):
    return pl.pallas_call(
        _tc_l1_body,
        grid=(2, N // _RB),
        in_specs=[
            pl.BlockSpec((_RB, 128), lambda j, r: (r, 0)),
            pl.BlockSpec((128, HALF), lambda j, r: (0, j)),
            pl.BlockSpec((NC, _RB, HALF), lambda j, r: (0, r, 0)),
        ],
        out_specs=[
            pl.BlockSpec((1, _RB, HALF), lambda j, r: (j, r, 0)),
            pl.BlockSpec((_RB, 16), lambda j, r: (r, 0)),
        ],
        out_shape=[
            jax.ShapeDtypeStruct((NC, N, HALF), jnp.float32),
            jax.ShapeDtypeStruct((N, 16), jnp.float32),
        ],
    )(x, W1, deg_parts)


def _tc_layer_body(s_ref, w_ref, dinv_ref, b_ref, out_ref):
    dinv = dinv_ref[:, 0:1]
    agg = jnp.concatenate([s_ref[0], s_ref[1]], axis=-1)
    a = jnp.maximum(dinv * agg + b_ref[...], 0.0)
    h = jnp.dot(a, w_ref[...], preferred_element_type=jnp.float32)
    out_ref[0] = dinv * h


def _tc_layer(s, W, b2d, dinv16):
    return pl.pallas_call(
        _tc_layer_body,
        grid=(2, N // _RB),
        in_specs=[
            pl.BlockSpec((NC, _RB, HALF), lambda j, r: (0, r, 0)),
            pl.BlockSpec((256, HALF), lambda j, r: (0, j)),
            pl.BlockSpec((_RB, 16), lambda j, r: (r, 0)),
            pl.BlockSpec((1, 256), lambda j, r: (0, 0)),
        ],
        out_specs=pl.BlockSpec((1, _RB, HALF), lambda j, r: (j, r, 0)),
        out_shape=jax.ShapeDtypeStruct((NC, N, HALF), jnp.float32),
    )(s, W, dinv16, b2d)


def _tc_scale_body(s_ref, dinv_ref, out_ref):
    dinv = dinv_ref[:, 0:1]
    out_ref[0] = dinv * s_ref[0]


def _tc_scale(s, dinv16):
    return pl.pallas_call(
        _tc_scale_body,
        grid=(2, N // _RB),
        in_specs=[
            pl.BlockSpec((1, _RB, HALF), lambda j, r: (j, r, 0)),
            pl.BlockSpec((_RB, 16), lambda j, r: (r, 0)),
        ],
        out_specs=pl.BlockSpec((1, _RB, HALF), lambda j, r: (j, r, 0)),
        out_shape=jax.ShapeDtypeStruct((NC, N, HALF), jnp.float32),
    )(s, dinv16)


def _tc_head_body(sums_ref, cnt_ref, b3_ref, wl_ref, bl_ref, out_ref):
    cnt = cnt_ref[:, 0:1]
    pooled = jnp.concatenate([sums_ref[0], sums_ref[1]], axis=-1)
    pooled = pooled / jnp.maximum(cnt, 1.0)
    pooled = jnp.where(cnt > 0.0, pooled + b3_ref[...], 0.0)
    out_ref[...] = jnp.dot(pooled, wl_ref[...],
                           preferred_element_type=jnp.float32) + bl_ref[...]


def _tc_head(sums, cnt, b3_2d, Wlin, blin2d):
    return pl.pallas_call(
        _tc_head_body,
        out_shape=jax.ShapeDtypeStruct((NUM_GRAPHS, 10), jnp.float32),
    )(sums, cnt, b3_2d, Wlin, blin2d)


@jax.jit
def _gcn(x, edge_index, batch, W1, b1, W2, b2, W3, b3, Wlin, blin):
    ei = edge_index.astype(jnp.int32)
    src2d = ei[0].reshape(ECHUNKS, EK)
    dst2d = ei[1].reshape(ECHUNKS, EK)
    tiler = jnp.clip(jnp.arange(NS)[:, None] * CT + jnp.arange(CTP)[None, :],
                     0, ECHUNKS - 1).reshape(-1)
    src_t = src2d[tiler].reshape(NS, CTP, EK)
    dst_t = dst2d[tiler].reshape(NS, CTP, EK)
    padd = jnp.zeros((DT * NC * NS - ECHUNKS, EK), jnp.int32)
    dstdeg_t = jnp.concatenate([dst2d, padd]).reshape(NC * NS, DT, EK)
    batch3d = batch.astype(jnp.int32).reshape(NCHUNKS, 1, NK)
    ones_in = jnp.ones((EK, HALF), jnp.float32)
    zeros_in = jnp.zeros((RTC, HALF), jnp.float32)
    zeros_pool = jnp.zeros((NUM_GRAPHS, HALF), jnp.float32)

    deg_parts, cnt = _sc_deg_cnt(dstdeg_t, batch3d, ones_in, zeros_in)
    hs1, dinv16 = _tc_l1(x, W1, deg_parts)
    s1 = _sc_conv(hs1, src_t, dst_t)
    hs2 = _tc_layer(s1, W2, b1.reshape(1, 256), dinv16)
    s2 = _sc_conv(hs2, src_t, dst_t)
    hs3 = _tc_layer(s2, W3, b2.reshape(1, 256), dinv16)
    s3 = _sc_conv(hs3, src_t, dst_t)
    h3 = _tc_scale(s3, dinv16)
    sums = _sc_pool(h3, batch3d, zeros_pool)
    return _tc_head(sums, cnt, b3.reshape(1, 256), Wlin, blin.reshape(1, 10))


def kernel(x, edge_index, batch, W1, b1, W2, b2, W3, b3, Wlin, blin):
    return _gcn(x, edge_index, batch, W1, b1, W2, b2, W3, b3, Wlin, blin)


# speedup vs baseline: 1.1471x; 1.0128x over previous
import functools

import jax
import jax.numpy as jnp
from jax import lax
from jax.experimental import pallas as pl
from jax.experimental.pallas import tpu as pltpu
from jax.experimental.pallas import tpu_sc as plsc

N = 10000
E = 320000
NUM_GRAPHS = 64
HALF = 128
NC, NS = 2, 16
EK = 128
ECHUNKS = E // EK
NK = 80
NCHUNKS = N // NK
RT0 = 624
RTC = 640

_mesh = plsc.VectorSubcoreMesh(
    core_axis_name="c", subcore_axis_name="s", num_cores=NC, num_subcores=NS)


DT = 80


@functools.partial(
    pl.kernel,
    out_type=(
        jax.ShapeDtypeStruct((NC, N, HALF), jnp.float32),
        jax.ShapeDtypeStruct((NUM_GRAPHS, HALF), jnp.float32),
    ),
    mesh=_mesh,
    scratch_types=[
        pltpu.VMEM_SHARED((N, HALF), jnp.float32),
        pltpu.VMEM_SHARED((NUM_GRAPHS, HALF), jnp.float32),
        pltpu.VMEM((EK, HALF), jnp.float32),
        pltpu.VMEM((NK, HALF), jnp.float32),
        pltpu.VMEM((DT, EK), jnp.int32),
        pltpu.VMEM((NK,), jnp.int32),
        pltpu.SemaphoreType.DMA,
        pltpu.SemaphoreType.DMA,
    ],
)
def _sc_deg_cnt(dstdeg_t, batch3d, ones_in, zeros_in, deg_out, cnt_out,
                deg_s, cnt_s, ones_e, ones_n, idx_e, idx_n, semA, semB):
    c = lax.axis_index("c")
    s = lax.axis_index("s")
    t = c * NS + s
    pltpu.sync_copy(ones_in, ones_e)
    nch = jnp.clip(ECHUNKS - t * DT, 0, DT)
    pltpu.sync_copy(dstdeg_t.at[t], idx_e)
    pltpu.sync_copy(zeros_in, deg_s.at[pl.ds(s * RT0, RTC)])

    @pl.when((c == 1) & (s == 0))
    def _():
        pltpu.sync_copy(zeros_in.at[pl.ds(0, NUM_GRAPHS)], cnt_s)

    plsc.subcore_barrier()

    def body(m, carry):
        kA = 2 * m
        dA = pltpu.async_copy(ones_e, deg_s.at[idx_e.at[kA]], semA, add=True)
        kB = kA + 1

        @pl.when(kB < nch)
        def _():
            dB = pltpu.async_copy(ones_e, deg_s.at[idx_e.at[kB]], semB,
                                  add=True)
            dA.wait()
            dB.wait()

        @pl.when(kB >= nch)
        def _():
            dA.wait()

        return carry

    lax.fori_loop(0, (nch + 1) // 2, body, 0)

    @pl.when(c == 1)
    def _():
        pltpu.sync_copy(ones_in.at[pl.ds(0, NK)], ones_n)
        nchn = (NCHUNKS - s + NS - 1) // NS

        def bodyn(k, carry):
            i = s + NS * k
            pltpu.sync_copy(batch3d.at[i, 0], idx_n)
            pltpu.sync_copy(ones_n, cnt_s.at[idx_n], add=True)
            return carry

        lax.fori_loop(0, nchn, bodyn, 0)

    plsc.subcore_barrier()
    pltpu.sync_copy(deg_s.at[pl.ds(s * RT0, RTC)],
                    deg_out.at[c].at[pl.ds(s * RT0, RTC)])

    @pl.when((c == 1) & (s == 0))
    def _():
        pltpu.sync_copy(cnt_s, cnt_out)


CT = 157
SEG = 32
NSEG = 5
CTP = SEG * NSEG


@functools.partial(
    pl.kernel,
    out_type=jax.ShapeDtypeStruct((NC, N, HALF), jnp.float32),
    mesh=_mesh,
    scratch_types=[
        pltpu.VMEM_SHARED((N, HALF), jnp.float32),
        pltpu.VMEM((EK, HALF), jnp.float32),
        pltpu.VMEM((EK, HALF), jnp.float32),
        pltpu.VMEM((SEG, EK), jnp.int32),
        pltpu.VMEM((SEG, EK), jnp.int32),
        pltpu.SemaphoreType.DMA,
        pltpu.SemaphoreType.DMA,
    ],
)
def _sc_conv(hs, src_t, dst_t, out, acc, rowsA, rowsB, isrc, idst,
             semA, semB):
    c = lax.axis_index("c")
    s = lax.axis_index("s")
    hs_c = hs.at[c]
    nch = jnp.minimum(CT, ECHUNKS - s * CT)
    pltpu.sync_copy(hs_c.at[pl.ds(s * RT0, RTC)], acc.at[pl.ds(s * RT0, RTC)])
    plsc.subcore_barrier()

    def seg_body(g, carry0):
        nseg = jnp.clip(nch - g * SEG, 0, SEG)

        @pl.when(nseg > 0)
        def _():
            pltpu.sync_copy(src_t.at[s].at[pl.ds(g * SEG, SEG)], isrc)
            pltpu.sync_copy(dst_t.at[s].at[pl.ds(g * SEG, SEG)], idst)

            def body(m, carry):
                kA = 2 * m
                gA = pltpu.async_copy(hs_c.at[isrc.at[kA]], rowsA, semA)

                @pl.when(kA >= 1)
                def _():
                    pltpu.sync_copy(rowsB, acc.at[idst.at[kA - 1]], add=True)

                gA.wait()
                kB = kA + 1

                @pl.when(kB < nseg)
                def _():
                    gB = pltpu.async_copy(hs_c.at[isrc.at[kB]], rowsB, semB)
                    pltpu.sync_copy(rowsA, acc.at[idst.at[kA]], add=True)
                    gB.wait()

                @pl.when(kB >= nseg)
                def _():
                    pltpu.sync_copy(rowsA, acc.at[idst.at[kA]], add=True)

                return carry

            lax.fori_loop(0, (nseg + 1) // 2, body, 0)

            @pl.when((nseg % 2 == 0) & (nseg >= 2))
            def _():
                pltpu.sync_copy(rowsB, acc.at[idst.at[nseg - 1]], add=True)

        return carry0

    lax.fori_loop(0, NSEG, seg_body, 0)
    plsc.subcore_barrier()
    pltpu.sync_copy(acc.at[pl.ds(s * RT0, RTC)],
                    out.at[c].at[pl.ds(s * RT0, RTC)])


@functools.partial(
    pl.kernel,
    out_type=jax.ShapeDtypeStruct((NC, NUM_GRAPHS, HALF), jnp.float32),
    mesh=_mesh,
    scratch_types=[
        pltpu.VMEM_SHARED((NUM_GRAPHS, HALF), jnp.float32),
        pltpu.VMEM((NK, HALF), jnp.float32),
        pltpu.VMEM((NK,), jnp.int32),
    ],
)
def _sc_pool(h3, batch3d, zeros_in, out, sums_s, rows, idx):
    c = lax.axis_index("c")
    s = lax.axis_index("s")
    h3_c = h3.at[c]

    @pl.when(s == 0)
    def _():
        pltpu.sync_copy(zeros_in, sums_s)

    plsc.subcore_barrier()
    nch = (NCHUNKS - s + NS - 1) // NS

    def body(k, carry):
        i = s + NS * k
        pltpu.sync_copy(batch3d.at[i, 0], idx)
        pltpu.sync_copy(h3_c.at[pl.ds(i * NK, NK)], rows)
        pltpu.sync_copy(rows, sums_s.at[idx], add=True)
        return carry

    lax.fori_loop(0, nch, body, 0)
    plsc.subcore_barrier()

    @pl.when(s == 0)
    def _():
        pltpu.sync_copy(sums_s, out.at[c])


_RB = 10000


def _tc_l1_body(x_ref, w_ref, deg_ref, out_ref, dinv_ref):
    deg = deg_ref[0, :, 0:1] + deg_ref[1, :, 0:1]
    dinv = lax.rsqrt(deg + 1.0)
    dinv_ref[...] = jnp.broadcast_to(dinv, (_RB, 16))
    h = jnp.dot(x_ref[...], w_ref[...], preferred_element_type=jnp.float32)
    out_ref[0] = dinv * h


def _tc_l1(x, W1, deg_parts):
    return pl.pallas_call(
        _tc_l1_body,
        grid=(2, N // _RB),
        in_specs=[
            pl.BlockSpec((_RB, 128), lambda j, r: (r, 0)),
            pl.BlockSpec((128, HALF), lambda j, r: (0, j)),
            pl.BlockSpec((NC, _RB, HALF), lambda j, r: (0, r, 0)),
        ],
        out_specs=[
            pl.BlockSpec((1, _RB, HALF), lambda j, r: (j, r, 0)),
            pl.BlockSpec((_RB, 16), lambda j, r: (r, 0)),
        ],
        out_shape=[
            jax.ShapeDtypeStruct((NC, N, HALF), jnp.float32),
            jax.ShapeDtypeStruct((N, 16), jnp.float32),
        ],
    )(x, W1, deg_parts)


def _tc_layer_body(s_ref, w_ref, dinv_ref, b_ref, out_ref):
    dinv = dinv_ref[:, 0:1]
    agg = jnp.concatenate([s_ref[0], s_ref[1]], axis=-1)
    a = jnp.maximum(dinv * agg + b_ref[...], 0.0)
    h = jnp.dot(a, w_ref[...], preferred_element_type=jnp.float32)
    out_ref[0] = dinv * h


def _tc_layer(s, W, b2d, dinv16):
    return pl.pallas_call(
        _tc_layer_body,
        grid=(2, N // _RB),
        in_specs=[
            pl.BlockSpec((NC, _RB, HALF), lambda j, r: (0, r, 0)),
            pl.BlockSpec((256, HALF), lambda j, r: (0, j)),
            pl.BlockSpec((_RB, 16), lambda j, r: (r, 0)),
            pl.BlockSpec((1, 256), lambda j, r: (0, 0)),
        ],
        out_specs=pl.BlockSpec((1, _RB, HALF), lambda j, r: (j, r, 0)),
        out_shape=jax.ShapeDtypeStruct((NC, N, HALF), jnp.float32),
    )(s, W, dinv16, b2d)


def _tc_scale_body(s_ref, dinv_ref, out_ref):
    dinv = dinv_ref[:, 0:1]
    out_ref[0] = dinv * s_ref[0]


def _tc_scale(s, dinv16):
    return pl.pallas_call(
        _tc_scale_body,
        grid=(2, N // _RB),
        in_specs=[
            pl.BlockSpec((1, _RB, HALF), lambda j, r: (j, r, 0)),
            pl.BlockSpec((_RB, 16), lambda j, r: (r, 0)),
        ],
        out_specs=pl.BlockSpec((1, _RB, HALF), lambda j, r: (j, r, 0)),
        out_shape=jax.ShapeDtypeStruct((NC, N, HALF), jnp.float32),
    )(s, dinv16)


def _tc_head_body(sums_ref, cnt_ref, b3_ref, wl_ref, bl_ref, out_ref):
    cnt = cnt_ref[:, 0:1]
    pooled = jnp.concatenate([sums_ref[0], sums_ref[1]], axis=-1)
    pooled = pooled / jnp.maximum(cnt, 1.0)
    pooled = jnp.where(cnt > 0.0, pooled + b3_ref[...], 0.0)
    out_ref[...] = jnp.dot(pooled, wl_ref[...],
                           preferred_element_type=jnp.float32) + bl_ref[...]


def _tc_head(sums, cnt, b3_2d, Wlin, blin2d):
    return pl.pallas_call(
        _tc_head_body,
        out_shape=jax.ShapeDtypeStruct((NUM_GRAPHS, 10), jnp.float32),
    )(sums, cnt, b3_2d, Wlin, blin2d)


@jax.jit
def _gcn(x, edge_index, batch, W1, b1, W2, b2, W3, b3, Wlin, blin):
    ei = edge_index.astype(jnp.int32)
    src2d = ei[0].reshape(ECHUNKS, EK)
    dst2d = ei[1].reshape(ECHUNKS, EK)
    tiler = jnp.clip(jnp.arange(NS)[:, None] * CT + jnp.arange(CTP)[None, :],
                     0, ECHUNKS - 1).reshape(-1)
    src_t = src2d[tiler].reshape(NS, CTP, EK)
    dst_t = dst2d[tiler].reshape(NS, CTP, EK)
    padd = jnp.zeros((DT * NC * NS - ECHUNKS, EK), jnp.int32)
    dstdeg_t = jnp.concatenate([dst2d, padd]).reshape(NC * NS, DT, EK)
    batch3d = batch.astype(jnp.int32).reshape(NCHUNKS, 1, NK)
    ones_in = jnp.ones((EK, HALF), jnp.float32)
    zeros_in = jnp.zeros((RTC, HALF), jnp.float32)
    zeros_pool = jnp.zeros((NUM_GRAPHS, HALF), jnp.float32)

    deg_parts, cnt = _sc_deg_cnt(dstdeg_t, batch3d, ones_in, zeros_in)
    hs1, dinv16 = _tc_l1(x, W1, deg_parts)
    s1 = _sc_conv(hs1, src_t, dst_t)
    hs2 = _tc_layer(s1, W2, b1.reshape(1, 256), dinv16)
    s2 = _sc_conv(hs2, src_t, dst_t)
    hs3 = _tc_layer(s2, W3, b2.reshape(1, 256), dinv16)
    s3 = _sc_conv(hs3, src_t, dst_t)
    h3 = _tc_scale(s3, dinv16)
    sums = _sc_pool(h3, batch3d, zeros_pool)
    return _tc_head(sums, cnt, b3.reshape(1, 256), Wlin, blin.reshape(1, 10))


def kernel(x, edge_index, batch, W1, b1, W2, b2, W3, b3, Wlin, blin):
    return _gcn(x, edge_index, batch, W1, b1, W2, b2, W3, b3, Wlin, blin)
